# Initial kernel scaffold; baseline (speedup 1.0000x reference)
import jax
import jax.numpy as jnp
from jax.experimental import pallas as pl


def kernel(x):
    raise NotImplementedError("write your pallas kernel here")


# speedup vs baseline: 57.8690x; 57.8690x over previous
import functools

import jax
import jax.numpy as jnp
from jax import lax
from jax.experimental import pallas as pl
from jax.experimental.pallas import tpu as pltpu
from jax.experimental.pallas import tpu_sc as plsc

N = 128
M = N * N
NCH = M // 16
HALF = 8 * M // 2
BIG = 1 << 20


def _two_wl_body(x_hbm, xt_hbm, out_hbm, xv, buf1, buf2, table, enc, ids):
    cid0 = lax.axis_index("c")
    sid0 = lax.axis_index("s")

    @pl.when((cid0 == 0) & (sid0 == 0))
    def _run():
        io = lax.iota(jnp.int32, 16)
        rio = 15 - io
        zero = jnp.zeros((16,), jnp.int32)

        pltpu.sync_copy(x_hbm, xv)
        pltpu.sync_copy(xt_hbm, buf1)

        def _hist(src_ref, dst_off):
            def bbody(b, _):
                def jbody(j, carry):
                    e1, e2 = carry
                    v = src_ref[pl.ds(j * N + b * 16, 16)]
                    inc = jnp.left_shift(jnp.int32(1), (v & 3) * 8)
                    lo = v < 4
                    return (e1 + jnp.where(lo, inc, zero),
                            e2 + jnp.where(lo, zero, inc))

                e1, e2 = lax.fori_loop(0, N, jbody, (zero, zero))
                enc[pl.ds(dst_off + b * 16, 16)] = e1
                enc[pl.ds(dst_off + N + b * 16, 16)] = e2
                return 0

            lax.fori_loop(0, 8, bbody, 0)

        _hist(buf1, 0)
        _hist(xv, 256)

        def _assign_ids(enc_off, dst_off):
            def bbody(b, _):
                e1v = enc[pl.ds(enc_off + b * 16, 16)]
                e2v = enc[pl.ds(enc_off + N + b * 16, 16)]

                def jbody(j, best):
                    jv = zero + j
                    a1 = plsc.load_gather(enc, [enc_off + jv])
                    a2 = plsc.load_gather(enc, [enc_off + N + jv])
                    eq = (e1v == a1) & (e2v == a2)
                    return jnp.where(eq, jnp.minimum(best, j), best)

                best = lax.fori_loop(0, N, jbody, jnp.full((16,), BIG, jnp.int32))
                ids[pl.ds(dst_off + b * 16, 16)] = best
                return 0

            lax.fori_loop(0, 8, bbody, 0)

        _assign_ids(0, 0)
        _assign_ids(256, N)

        def kbody(v1, _):
            rterm = plsc.load_gather(ids, [zero + v1]) * N

            def cbody(jc, _):
                base = v1 * N + jc * 16
                k = xv[pl.ds(base, 16)] * M + rterm + ids[pl.ds(N + jc * 16, 16)]
                buf2[pl.ds(base, 16)] = k
                return 0

            lax.fori_loop(0, 8, cbody, 0)
            return 0

        lax.fori_loop(0, N, kbody, 0)

        for p in (1, 0):
            lo = jnp.int32(p * HALF)

            def sbody(t, _, lo=lo):
                base = (NCH - 1 - t) * 16
                kr = lax.rev(buf2[pl.ds(base, 16)], (0,))
                inb = (kr >= lo) & (kr < lo + HALF)
                _, last = plsc.scan_count(kr, mask=inb)
                idx = jnp.where(inb, kr - lo, 0)
                plsc.store_scatter(table, [idx], base + rio, mask=last & inb)
                return 0

            lax.fori_loop(0, NCH, sbody, 0)

            def gbody(t, _, lo=lo, first_pass=(p == 1)):
                base = t * 16
                k = buf2[pl.ds(base, 16)]
                inb = (k >= lo) & (k < lo + HALF)
                f = plsc.load_gather(table, [jnp.where(inb, k - lo, 0)],
                                     mask=inb)
                prev = zero if first_pass else buf1[pl.ds(base, 16)]
                buf1[pl.ds(base, 16)] = jnp.where(inb, f, prev)
                return 0

            lax.fori_loop(0, NCH, gbody, 0)

        def ebody(t, c):
            base = t * 16
            isf = jnp.where(buf1[pl.ds(base, 16)] == base + io, 1, 0)
            buf2[pl.ds(base, 16)] = plsc.cumsum(isf) + c
            return c + jnp.sum(isf)

        lax.fori_loop(0, NCH, ebody, jnp.int32(0))

        def fbody(t, _):
            base = t * 16
            f = buf1[pl.ds(base, 16)]
            buf1[pl.ds(base, 16)] = plsc.load_gather(buf2, [f]) - 1
            return 0

        lax.fori_loop(0, NCH, fbody, 0)

        pltpu.sync_copy(buf1, out_hbm)


@jax.jit
def kernel(x):
    xi = x.astype(jnp.int32)
    run = pl.kernel(
        _two_wl_body,
        out_type=jax.ShapeDtypeStruct((M,), jnp.int32),
        mesh=plsc.VectorSubcoreMesh(core_axis_name="c", subcore_axis_name="s"),
        compiler_params=pltpu.CompilerParams(needs_layout_passes=False),
        scratch_types=[
            pltpu.VMEM((M,), jnp.int32),
            pltpu.VMEM((M,), jnp.int32),
            pltpu.VMEM((M,), jnp.int32),
            pltpu.VMEM((HALF,), jnp.int32),
            pltpu.VMEM((512,), jnp.int32),
            pltpu.VMEM((256,), jnp.int32),
        ],
    )
    out = run(xi.reshape(M), xi.T.reshape(M))
    return out.reshape(N, N).astype(jnp.int64)


# speedup vs baseline: 143.9689x; 2.4878x over previous
import functools

import jax
import jax.numpy as jnp
from jax import lax
from jax.experimental import pallas as pl
from jax.experimental.pallas import tpu as pltpu
from jax.experimental.pallas import tpu_sc as plsc

N = 128
M = N * N
RK = 8 * N
KS = N * RK
BIG = 1 << 20


def _two_wl_body(x_hbm, out_hbm,
                 xv, enc, ids, rfloc, kb, fb, rb, csb, accs, stage, st16,
                 enc_sh, ids_sh, rf_sh, g_sh, tot_sh, csum_sh):
    core = lax.axis_index("c")
    sid = lax.axis_index("s")
    io = lax.iota(jnp.int32, 16)
    rio = 15 - io
    zero = jnp.zeros((16,), jnp.int32)

    pltpu.sync_copy(x_hbm, xv)

    def _acc(carry, v):
        e1, e2 = carry
        inc = jnp.left_shift(jnp.int32(1), (v & 3) * 8)
        lo = v < 4
        return (e1 + jnp.where(lo, inc, zero), e2 + jnp.where(lo, zero, inc))

    @pl.when(sid < 8)
    def _rows():
        stride_idx = io * N + sid * 16 * N

        def jbody(j, carry):
            return _acc(carry, plsc.load_gather(xv, [stride_idx + j]))

        e1, e2 = lax.fori_loop(0, N, jbody, (zero, zero))
        enc[pl.ds(sid * 16, 16)] = e1
        enc[pl.ds(N + sid * 16, 16)] = e2
        pltpu.sync_copy(enc.at[pl.ds(sid * 16, 16)],
                        enc_sh.at[pl.ds(sid * 16, 16)])
        pltpu.sync_copy(enc.at[pl.ds(N + sid * 16, 16)],
                        enc_sh.at[pl.ds(N + sid * 16, 16)])

    @pl.when(sid >= 8)
    def _cols():
        c0 = (sid - 8) * 16

        def jbody(j, carry):
            return _acc(carry, xv[pl.ds(j * N + c0, 16)])

        e1, e2 = lax.fori_loop(0, N, jbody, (zero, zero))
        enc[pl.ds(256 + c0, 16)] = e1
        enc[pl.ds(256 + N + c0, 16)] = e2
        pltpu.sync_copy(enc.at[pl.ds(256 + c0, 16)],
                        enc_sh.at[pl.ds(256 + c0, 16)])
        pltpu.sync_copy(enc.at[pl.ds(256 + N + c0, 16)],
                        enc_sh.at[pl.ds(256 + N + c0, 16)])

    plsc.subcore_barrier()
    pltpu.sync_copy(enc_sh, enc)

    def _assign(enc_off, b, dst_off):
        e1v = enc[pl.ds(enc_off + b * 16, 16)]
        e2v = enc[pl.ds(enc_off + N + b * 16, 16)]

        def jbody(j, best):
            jv = zero + j
            a1 = plsc.load_gather(enc, [enc_off + jv])
            a2 = plsc.load_gather(enc, [enc_off + N + jv])
            eq = (e1v == a1) & (e2v == a2)
            return jnp.where(eq, jnp.minimum(best, j), best)

        best = lax.fori_loop(0, N, jbody, jnp.full((16,), BIG, jnp.int32))
        ids[pl.ds(dst_off + b * 16, 16)] = best
        pltpu.sync_copy(ids.at[pl.ds(dst_off + b * 16, 16)],
                        ids_sh.at[pl.ds(dst_off + b * 16, 16)])

    @pl.when(sid < 8)
    def _rid():
        _assign(0, sid, 0)

    @pl.when(sid >= 8)
    def _cid():
        _assign(256, sid - 8, N)

    plsc.subcore_barrier()
    pltpu.sync_copy(ids_sh, ids)

    pc = [ids[pl.ds(N + jc * 16, 16)] for jc in range(8)]

    def rowbody(r8, _):
        v1 = sid * 8 + r8

        def ibody(q, _):
            rfloc[pl.ds(q * 16, 16)] = zero + BIG
            return 0

        lax.fori_loop(0, RK // 16, ibody, 0)
        for jc in range(7, -1, -1):
            rk = xv[pl.ds(v1 * N + jc * 16, 16)] * N + pc[jc]
            rkr = lax.rev(rk, (0,))
            _, last = plsc.scan_count(rkr)
            plsc.store_scatter(rfloc, [rkr], jc * 16 + rio, mask=last)
        pltpu.sync_copy(rfloc, rf_sh.at[pl.ds(v1 * RK, RK)])
        return 0

    lax.fori_loop(0, 8, rowbody, 0)
    plsc.subcore_barrier()

    lo_r = sid * 8

    def dinit(q, _):
        accs[pl.ds(q * 16, 16)] = zero + BIG
        return 0

    lax.fori_loop(0, 8 * RK // 16, dinit, 0)

    def dscan(v1, _):
        rv = plsc.load_gather(ids, [zero + v1])
        r = rv[0]

        @pl.when((r >= lo_r) & (r < lo_r + 8))
        def _merge():
            pltpu.sync_copy(rf_sh.at[pl.ds(v1 * RK, RK)], stage)
            base = (r - lo_r) * RK

            def mbody(q, _):
                a = accs[pl.ds(base + q * 16, 16)]
                s = stage[pl.ds(q * 16, 16)] + v1 * N
                accs[pl.ds(base + q * 16, 16)] = jnp.minimum(a, s)
                return 0

            lax.fori_loop(0, RK // 16, mbody, 0)

        return 0

    lax.fori_loop(0, N, dscan, 0)

    def dpub(g8, _):
        pltpu.sync_copy(accs.at[pl.ds(g8 * RK, RK)],
                        g_sh.at[pl.ds((lo_r + g8) * RK, RK)])
        return 0

    lax.fori_loop(0, 8, dpub, 0)
    plsc.subcore_barrier()

    def kbuild(r8, _):
        v1 = sid * 8 + r8
        rterm = plsc.load_gather(ids, [zero + v1]) * RK
        for jc in range(8):
            k = xv[pl.ds(v1 * N + jc * 16, 16)] * N + pc[jc] + rterm
            kb[pl.ds(r8 * N + jc * 16, 16)] = k
        return 0

    lax.fori_loop(0, 8, kbuild, 0)
    for j in range(8):
        pltpu.sync_copy(g_sh.at[kb.at[pl.ds(j * 128, 128)]],
                        fb.at[pl.ds(j * 128, 128)])

    base0 = sid * 1024

    def tbody(q, c):
        isf = jnp.where(fb[pl.ds(q * 16, 16)] == base0 + q * 16 + io, 1, 0)
        return c + jnp.sum(isf)

    tot = lax.fori_loop(0, 64, tbody, jnp.int32(0))
    st16[pl.ds(0, 16)] = zero + tot
    pltpu.sync_copy(st16, tot_sh.at[pl.ds(sid * 16, 16)])
    plsc.subcore_barrier()
    pltpu.sync_copy(tot_sh, stage.at[pl.ds(0, 256)])
    tvec = plsc.load_gather(stage, [io * 16])
    off = jnp.sum(jnp.where(io < sid, tvec, 0))

    def cbody(q, c):
        isf = jnp.where(fb[pl.ds(q * 16, 16)] == base0 + q * 16 + io, 1, 0)
        csb[pl.ds(q * 16, 16)] = plsc.cumsum(isf) + c
        return c + jnp.sum(isf)

    lax.fori_loop(0, 64, cbody, off)
    pltpu.sync_copy(csb, csum_sh.at[pl.ds(base0, 1024)])
    plsc.subcore_barrier()

    for j in range(8):
        pltpu.sync_copy(csum_sh.at[fb.at[pl.ds(j * 128, 128)]],
                        rb.at[pl.ds(j * 128, 128)])

    def obody(q, _):
        csb[pl.ds(q * 16, 16)] = rb[pl.ds(q * 16, 16)] - 1
        return 0

    lax.fori_loop(0, 64, obody, 0)

    @pl.when(core == 0)
    def _write():
        pltpu.sync_copy(csb, out_hbm.at[pl.ds(base0, 1024)])


@jax.jit
def kernel(x):
    xi = x.astype(jnp.int32)
    run = pl.kernel(
        _two_wl_body,
        out_type=jax.ShapeDtypeStruct((M,), jnp.int32),
        mesh=plsc.VectorSubcoreMesh(core_axis_name="c", subcore_axis_name="s"),
        compiler_params=pltpu.CompilerParams(needs_layout_passes=False),
        scratch_types=[
            pltpu.VMEM((M,), jnp.int32),
            pltpu.VMEM((512,), jnp.int32),
            pltpu.VMEM((256,), jnp.int32),
            pltpu.VMEM((RK,), jnp.int32),
            pltpu.VMEM((1024,), jnp.int32),
            pltpu.VMEM((1024,), jnp.int32),
            pltpu.VMEM((1024,), jnp.int32),
            pltpu.VMEM((1024,), jnp.int32),
            pltpu.VMEM((8 * RK,), jnp.int32),
            pltpu.VMEM((RK,), jnp.int32),
            pltpu.VMEM((16,), jnp.int32),
            pltpu.VMEM_SHARED((512,), jnp.int32),
            pltpu.VMEM_SHARED((256,), jnp.int32),
            pltpu.VMEM_SHARED((N * RK,), jnp.int32),
            pltpu.VMEM_SHARED((KS,), jnp.int32),
            pltpu.VMEM_SHARED((256,), jnp.int32),
            pltpu.VMEM_SHARED((M,), jnp.int32),
        ],
    )
    out = run(xi.reshape(M))
    return out.reshape(N, N).astype(jnp.int64)


# speedup vs baseline: 164.1690x; 1.1403x over previous
import functools

import jax
import jax.numpy as jnp
from jax import lax
from jax.experimental import pallas as pl
from jax.experimental.pallas import tpu as pltpu
from jax.experimental.pallas import tpu_sc as plsc

N = 128
M = N * N
RK = 8 * N
KS = N * RK
BIG = 1 << 20


def _two_wl_body(x_hbm, out_hbm,
                 xv, enc, ids, rfbuf, kb, fb, rb, csb, accs, stage, st16,
                 enc_sh, ids_sh, rf_sh, g_sh, tot_sh, csum_sh, sem):
    core = lax.axis_index("c")
    sid = lax.axis_index("s")
    io = lax.iota(jnp.int32, 16)
    rio = 15 - io
    zero = jnp.zeros((16,), jnp.int32)

    pltpu.sync_copy(x_hbm, xv)

    def _acc(carry, v):
        e1, e2 = carry
        inc = jnp.left_shift(jnp.int32(1), (v & 3) * 8)
        lo = v < 4
        return (e1 + jnp.where(lo, inc, zero), e2 + jnp.where(lo, zero, inc))

    @pl.when(sid < 8)
    def _rows():
        stride_idx = io * N + sid * 16 * N

        def jbody(j, carry):
            return _acc(carry, plsc.load_gather(xv, [stride_idx + j]))

        e1, e2 = lax.fori_loop(0, N, jbody, (zero, zero))
        enc[pl.ds(sid * 16, 16)] = e1
        enc[pl.ds(N + sid * 16, 16)] = e2
        pltpu.sync_copy(enc.at[pl.ds(sid * 16, 16)],
                        enc_sh.at[pl.ds(sid * 16, 16)])
        pltpu.sync_copy(enc.at[pl.ds(N + sid * 16, 16)],
                        enc_sh.at[pl.ds(N + sid * 16, 16)])

    @pl.when(sid >= 8)
    def _cols():
        c0 = (sid - 8) * 16

        def jbody(j, carry):
            return _acc(carry, xv[pl.ds(j * N + c0, 16)])

        e1, e2 = lax.fori_loop(0, N, jbody, (zero, zero))
        enc[pl.ds(256 + c0, 16)] = e1
        enc[pl.ds(256 + N + c0, 16)] = e2
        pltpu.sync_copy(enc.at[pl.ds(256 + c0, 16)],
                        enc_sh.at[pl.ds(256 + c0, 16)])
        pltpu.sync_copy(enc.at[pl.ds(256 + N + c0, 16)],
                        enc_sh.at[pl.ds(256 + N + c0, 16)])

    plsc.subcore_barrier()
    pltpu.sync_copy(enc_sh, enc)

    def _assign(enc_off, b, dst_off):
        e1v = enc[pl.ds(enc_off + b * 16, 16)]
        e2v = enc[pl.ds(enc_off + N + b * 16, 16)]

        def jbody(j, best):
            jv = zero + j
            a1 = plsc.load_gather(enc, [enc_off + jv])
            a2 = plsc.load_gather(enc, [enc_off + N + jv])
            eq = (e1v == a1) & (e2v == a2)
            return jnp.where(eq, jnp.minimum(best, j), best)

        best = lax.fori_loop(0, N, jbody, jnp.full((16,), BIG, jnp.int32))
        ids[pl.ds(dst_off + b * 16, 16)] = best
        pltpu.sync_copy(ids.at[pl.ds(dst_off + b * 16, 16)],
                        ids_sh.at[pl.ds(dst_off + b * 16, 16)])

    @pl.when(sid < 8)
    def _rid():
        _assign(0, sid, 0)

    @pl.when(sid >= 8)
    def _cid():
        _assign(256, sid - 8, N)

    plsc.subcore_barrier()
    pltpu.sync_copy(ids_sh, ids)

    pc = [ids[pl.ds(N + jc * 16, 16)] for jc in range(8)]

    def cinit(q, _):
        rfbuf[pl.ds(q * 16, 16)] = zero + BIG
        return 0

    lax.fori_loop(0, 8 * RK // 16, cinit, 0)
    rf_dma = []
    for r8 in range(8):
        v1 = sid * 8 + r8
        for jc in range(7, -1, -1):
            rk = xv[pl.ds(v1 * N + jc * 16, 16)] * N + pc[jc] + r8 * RK
            rkr = lax.rev(rk, (0,))
            _, last = plsc.scan_count(rkr)
            plsc.store_scatter(rfbuf, [rkr], jc * 16 + rio, mask=last)
        rf_dma.append(pltpu.async_copy(rfbuf.at[pl.ds(r8 * RK, RK)],
                                       rf_sh.at[pl.ds(v1 * RK, RK)], sem))

    def dinit(q, _):
        g8 = q // (RK // 16)
        accs[pl.ds(q * 16, 16)] = (rfbuf[pl.ds(q * 16, 16)]
                                   + (sid * 8 + g8) * N)
        return 0

    lax.fori_loop(0, 8 * RK // 16, dinit, 0)
    for h in rf_dma:
        h.wait()
    plsc.subcore_barrier()

    lo_r = sid * 8

    def dscan(v1, _):
        rv = plsc.load_gather(ids, [zero + v1])
        r = rv[0]

        @pl.when((r >= lo_r) & (r < lo_r + 8) & (r != v1))
        def _merge():
            pltpu.sync_copy(rf_sh.at[pl.ds(v1 * RK, RK)], stage)
            base = (r - lo_r) * RK

            def mbody(q, _):
                a = accs[pl.ds(base + q * 16, 16)]
                s = stage[pl.ds(q * 16, 16)] + v1 * N
                accs[pl.ds(base + q * 16, 16)] = jnp.minimum(a, s)
                return 0

            lax.fori_loop(0, RK // 16, mbody, 0)

        return 0

    lax.fori_loop(0, N, dscan, 0)

    g_dma = [pltpu.async_copy(accs.at[pl.ds(g8 * RK, RK)],
                              g_sh.at[pl.ds((lo_r + g8) * RK, RK)], sem)
             for g8 in range(8)]
    for h in g_dma:
        h.wait()
    plsc.subcore_barrier()

    def kbuild(r8, _):
        v1 = sid * 8 + r8
        rterm = plsc.load_gather(ids, [zero + v1]) * RK
        for jc in range(8):
            k = xv[pl.ds(v1 * N + jc * 16, 16)] * N + pc[jc] + rterm
            kb[pl.ds(r8 * N + jc * 16, 16)] = k
        return 0

    lax.fori_loop(0, 8, kbuild, 0)
    f_dma = [pltpu.async_copy(g_sh.at[kb.at[pl.ds(j * 128, 128)]],
                              fb.at[pl.ds(j * 128, 128)], sem)
             for j in range(8)]
    for h in f_dma:
        h.wait()

    base0 = sid * 1024

    def tbody(q, c):
        isf = jnp.where(fb[pl.ds(q * 16, 16)] == base0 + q * 16 + io, 1, 0)
        return c + jnp.sum(isf)

    tot = lax.fori_loop(0, 64, tbody, jnp.int32(0))
    st16[pl.ds(0, 16)] = zero + tot
    pltpu.sync_copy(st16, tot_sh.at[pl.ds(sid * 16, 16)])
    plsc.subcore_barrier()
    pltpu.sync_copy(tot_sh, stage.at[pl.ds(0, 256)])
    tvec = plsc.load_gather(stage, [io * 16])
    off = jnp.sum(jnp.where(io < sid, tvec, 0))

    def cbody(q, c):
        isf = jnp.where(fb[pl.ds(q * 16, 16)] == base0 + q * 16 + io, 1, 0)
        csb[pl.ds(q * 16, 16)] = plsc.cumsum(isf) + c
        return c + jnp.sum(isf)

    lax.fori_loop(0, 64, cbody, off)
    pltpu.sync_copy(csb, csum_sh.at[pl.ds(base0, 1024)])
    plsc.subcore_barrier()

    r_dma = [pltpu.async_copy(csum_sh.at[fb.at[pl.ds(j * 128, 128)]],
                              rb.at[pl.ds(j * 128, 128)], sem)
             for j in range(8)]
    for h in r_dma:
        h.wait()

    def obody(q, _):
        csb[pl.ds(q * 16, 16)] = rb[pl.ds(q * 16, 16)] - 1
        return 0

    lax.fori_loop(0, 64, obody, 0)

    @pl.when(core == 0)
    def _write():
        pltpu.sync_copy(csb, out_hbm.at[pl.ds(base0, 1024)])


@jax.jit
def kernel(x):
    xi = x.astype(jnp.int32)
    run = pl.kernel(
        _two_wl_body,
        out_type=jax.ShapeDtypeStruct((M,), jnp.int32),
        mesh=plsc.VectorSubcoreMesh(core_axis_name="c", subcore_axis_name="s"),
        compiler_params=pltpu.CompilerParams(needs_layout_passes=False),
        scratch_types=[
            pltpu.VMEM((M,), jnp.int32),
            pltpu.VMEM((512,), jnp.int32),
            pltpu.VMEM((256,), jnp.int32),
            pltpu.VMEM((8 * RK,), jnp.int32),
            pltpu.VMEM((1024,), jnp.int32),
            pltpu.VMEM((1024,), jnp.int32),
            pltpu.VMEM((1024,), jnp.int32),
            pltpu.VMEM((1024,), jnp.int32),
            pltpu.VMEM((8 * RK,), jnp.int32),
            pltpu.VMEM((RK,), jnp.int32),
            pltpu.VMEM((16,), jnp.int32),
            pltpu.VMEM_SHARED((512,), jnp.int32),
            pltpu.VMEM_SHARED((256,), jnp.int32),
            pltpu.VMEM_SHARED((N * RK,), jnp.int32),
            pltpu.VMEM_SHARED((KS,), jnp.int32),
            pltpu.VMEM_SHARED((256,), jnp.int32),
            pltpu.VMEM_SHARED((M,), jnp.int32),
            pltpu.SemaphoreType.DMA,
        ],
    )
    out = run(xi.reshape(M))
    return out.reshape(N, N).astype(jnp.int64)


# speedup vs baseline: 179.4376x; 1.0930x over previous
import functools

import jax
import jax.numpy as jnp
from jax import lax
from jax.experimental import pallas as pl
from jax.experimental.pallas import tpu as pltpu
from jax.experimental.pallas import tpu_sc as plsc

N = 128
M = N * N
RK = 8 * N
KS = N * RK
BIG = 1 << 20


def _two_wl_body(x_hbm, out_hbm,
                 xv, enc, ids, rfbuf, kb, fb, rb, csb, stage, st16,
                 enc_sh, ids_sh, rf_sh, g_sh, tot_sh, csum_sh, sem):
    core = lax.axis_index("c")
    sid = lax.axis_index("s")
    io = lax.iota(jnp.int32, 16)
    rio = 15 - io
    zero = jnp.zeros((16,), jnp.int32)

    pltpu.sync_copy(x_hbm, xv)

    def _acc(carry, vf):
        e1, e2 = carry
        v = vf.astype(jnp.int32)
        inc = jnp.left_shift(jnp.int32(1), (v & 3) * 8)
        lo = v < 4
        return (e1 + jnp.where(lo, inc, zero), e2 + jnp.where(lo, zero, inc))

    @pl.when(sid < 8)
    def _rows():
        stride_idx = io * N + sid * 16 * N

        def jbody(jj, carry):
            for t in range(4):
                carry = _acc(carry, plsc.load_gather(xv, [stride_idx + (jj * 4 + t)]))
            return carry

        e1, e2 = lax.fori_loop(0, N // 4, jbody, (zero, zero))
        enc[pl.ds(sid * 16, 16)] = e1
        enc[pl.ds(N + sid * 16, 16)] = e2
        pltpu.sync_copy(enc.at[pl.ds(sid * 16, 16)],
                        enc_sh.at[pl.ds(sid * 16, 16)])
        pltpu.sync_copy(enc.at[pl.ds(N + sid * 16, 16)],
                        enc_sh.at[pl.ds(N + sid * 16, 16)])

    @pl.when(sid >= 8)
    def _cols():
        c0 = (sid - 8) * 16

        def jbody(jj, carry):
            for t in range(4):
                carry = _acc(carry, xv[pl.ds((jj * 4 + t) * N + c0, 16)])
            return carry

        e1, e2 = lax.fori_loop(0, N // 4, jbody, (zero, zero))
        enc[pl.ds(256 + c0, 16)] = e1
        enc[pl.ds(256 + N + c0, 16)] = e2
        pltpu.sync_copy(enc.at[pl.ds(256 + c0, 16)],
                        enc_sh.at[pl.ds(256 + c0, 16)])
        pltpu.sync_copy(enc.at[pl.ds(256 + N + c0, 16)],
                        enc_sh.at[pl.ds(256 + N + c0, 16)])

    plsc.subcore_barrier()
    pltpu.sync_copy(enc_sh, enc)

    def _assign(enc_off, b, dst_off):
        e1v = enc[pl.ds(enc_off + b * 16, 16)]
        e2v = enc[pl.ds(enc_off + N + b * 16, 16)]

        def jbody(jj, best):
            for t in range(2):
                j = jj * 2 + t
                jv = zero + j
                a1 = plsc.load_gather(enc, [enc_off + jv])
                a2 = plsc.load_gather(enc, [enc_off + N + jv])
                eq = (e1v == a1) & (e2v == a2)
                best = jnp.where(eq, jnp.minimum(best, j), best)
            return best

        best = lax.fori_loop(0, N // 2, jbody, jnp.full((16,), BIG, jnp.int32))
        ids[pl.ds(dst_off + b * 16, 16)] = best
        pltpu.sync_copy(ids.at[pl.ds(dst_off + b * 16, 16)],
                        ids_sh.at[pl.ds(dst_off + b * 16, 16)])

    @pl.when(sid < 8)
    def _rid():
        _assign(0, sid, 0)

    @pl.when(sid >= 8)
    def _cid():
        _assign(256, sid - 8, N)

    plsc.subcore_barrier()
    pltpu.sync_copy(ids_sh, ids)

    pc = [ids[pl.ds(N + jc * 16, 16)] for jc in range(8)]

    def cinit(q, _):
        for t in range(4):
            rfbuf[pl.ds(q * 64 + t * 16, 16)] = zero + BIG
        return 0

    lax.fori_loop(0, 8 * RK // 64, cinit, 0)
    rf_dma = []
    for r8 in range(8):
        v1 = sid * 8 + r8
        for jc in range(7, -1, -1):
            xc = xv[pl.ds(v1 * N + jc * 16, 16)].astype(jnp.int32)
            rkr = lax.rev(xc * N + pc[jc], (0,)) + r8 * RK
            _, last = plsc.scan_count(rkr)
            plsc.store_scatter(rfbuf, [rkr], v1 * N + jc * 16 + rio,
                               mask=last)
        rf_dma.append(pltpu.async_copy(rfbuf.at[pl.ds(r8 * RK, RK)],
                                       rf_sh.at[pl.ds(v1 * RK, RK)], sem))
    for h in rf_dma:
        h.wait()
    plsc.subcore_barrier()

    lo_r = sid * 8

    def dscan(v1, _):
        rv = plsc.load_gather(ids, [zero + v1])
        r = rv[0]

        @pl.when((r >= lo_r) & (r < lo_r + 8) & (r != v1))
        def _merge():
            pltpu.sync_copy(rf_sh.at[pl.ds(v1 * RK, RK)], stage)
            base = (r - lo_r) * RK

            def mbody(q, _):
                a = rfbuf[pl.ds(base + q * 16, 16)]
                rfbuf[pl.ds(base + q * 16, 16)] = jnp.minimum(
                    a, stage[pl.ds(q * 16, 16)])
                return 0

            lax.fori_loop(0, RK // 16, mbody, 0)

        return 0

    lax.fori_loop(0, N, dscan, 0)

    g_dma = [pltpu.async_copy(rfbuf.at[pl.ds(g8 * RK, RK)],
                              g_sh.at[pl.ds((lo_r + g8) * RK, RK)], sem)
             for g8 in range(8)]
    for h in g_dma:
        h.wait()
    plsc.subcore_barrier()

    def kbuild(r8, _):
        v1 = sid * 8 + r8
        rterm = plsc.load_gather(ids, [zero + v1]) * RK
        for jc in range(8):
            xc = xv[pl.ds(v1 * N + jc * 16, 16)].astype(jnp.int32)
            kb[pl.ds(r8 * N + jc * 16, 16)] = xc * N + pc[jc] + rterm
        return 0

    lax.fori_loop(0, 8, kbuild, 0)
    f_dma = [pltpu.async_copy(g_sh.at[kb.at[pl.ds(j * 128, 128)]],
                              fb.at[pl.ds(j * 128, 128)], sem)
             for j in range(8)]
    for h in f_dma:
        h.wait()

    base0 = sid * 1024

    def cbody(q, c):
        isf = jnp.where(fb[pl.ds(q * 16, 16)] == base0 + q * 16 + io, 1, 0)
        csb[pl.ds(q * 16, 16)] = plsc.cumsum(isf) + c
        return c + jnp.sum(isf)

    tot = lax.fori_loop(0, 64, cbody, jnp.int32(0))
    st16[pl.ds(0, 16)] = zero + tot
    d1 = pltpu.async_copy(st16, tot_sh.at[pl.ds(sid * 16, 16)], sem)
    d2 = pltpu.async_copy(csb, csum_sh.at[pl.ds(base0, 1024)], sem)
    d1.wait()
    d2.wait()
    plsc.subcore_barrier()
    pltpu.sync_copy(tot_sh, stage.at[pl.ds(0, 256)])
    tvec = plsc.load_gather(stage, [io * 16])
    st16[pl.ds(0, 16)] = plsc.cumsum(tvec) - tvec

    r_dma = [pltpu.async_copy(csum_sh.at[fb.at[pl.ds(j * 128, 128)]],
                              rb.at[pl.ds(j * 128, 128)], sem)
             for j in range(8)]
    for h in r_dma:
        h.wait()

    def obody(q, _):
        blk = jnp.right_shift(fb[pl.ds(q * 16, 16)], 10)
        offc = plsc.load_gather(st16, [blk])
        csb[pl.ds(q * 16, 16)] = rb[pl.ds(q * 16, 16)] + offc - 1
        return 0

    lax.fori_loop(0, 64, obody, 0)

    @pl.when(core == 0)
    def _write():
        pltpu.sync_copy(csb, out_hbm.at[pl.ds(base0, 1024)])


@jax.jit
def kernel(x):
    run = pl.kernel(
        _two_wl_body,
        out_type=jax.ShapeDtypeStruct((M,), jnp.int32),
        mesh=plsc.VectorSubcoreMesh(core_axis_name="c", subcore_axis_name="s"),
        compiler_params=pltpu.CompilerParams(needs_layout_passes=False),
        scratch_types=[
            pltpu.VMEM((M,), jnp.float32),
            pltpu.VMEM((512,), jnp.int32),
            pltpu.VMEM((256,), jnp.int32),
            pltpu.VMEM((8 * RK,), jnp.int32),
            pltpu.VMEM((1024,), jnp.int32),
            pltpu.VMEM((1024,), jnp.int32),
            pltpu.VMEM((1024,), jnp.int32),
            pltpu.VMEM((1024,), jnp.int32),
            pltpu.VMEM((RK,), jnp.int32),
            pltpu.VMEM((16,), jnp.int32),
            pltpu.VMEM_SHARED((512,), jnp.int32),
            pltpu.VMEM_SHARED((256,), jnp.int32),
            pltpu.VMEM_SHARED((N * RK,), jnp.int32),
            pltpu.VMEM_SHARED((KS,), jnp.int32),
            pltpu.VMEM_SHARED((256,), jnp.int32),
            pltpu.VMEM_SHARED((M,), jnp.int32),
            pltpu.SemaphoreType.DMA,
        ],
    )
    out = run(x.reshape(M))
    return out.reshape(N, N).astype(jnp.int64)
